# Initial kernel scaffold; baseline (speedup 1.0000x reference)
import jax
import jax.numpy as jnp
from jax.experimental import pallas as pl


def kernel(x, batch):
    raise NotImplementedError("write your pallas kernel here")


# speedup vs baseline: 5.4792x; 5.4792x over previous
import jax
import jax.numpy as jnp
from jax import lax
from jax.experimental import pallas as pl
from jax.experimental.pallas import tpu as pltpu
from jax.experimental.pallas import tpu_sc as plsc

N_ROWS = 320000
D = 128
NUM_SEG = 1024
NC = 2
NS = 16
NW = NC * NS
ROWS_PER_W = N_ROWS // NW
CHUNK = 80
CHUNKS_PER_W = ROWS_PER_W // CHUNK
SEG_PER_TILE = NUM_SEG // NS
CNT_W = 16


def _sc_partial_sums(x, idx2d, zeros, zeros_c, ones):
    mesh = plsc.VectorSubcoreMesh(core_axis_name="c", subcore_axis_name="s")

    def body(x_hbm, idx_hbm, zeros_hbm, zeros_c_hbm, ones_hbm, psums_hbm,
             pcnts_hbm, idx_v, rows_v, ones_v, sums_sh, cnts_sh):
        cid = lax.axis_index("c")
        sid = lax.axis_index("s")
        wid = cid * NS + sid

        seg0 = sid * SEG_PER_TILE
        pltpu.sync_copy(zeros_hbm.at[pl.ds(seg0, SEG_PER_TILE), :],
                        sums_sh.at[pl.ds(seg0, SEG_PER_TILE), :])
        pltpu.sync_copy(zeros_c_hbm.at[pl.ds(seg0, SEG_PER_TILE), :],
                        cnts_sh.at[pl.ds(seg0, SEG_PER_TILE), :])

        pltpu.sync_copy(idx_hbm.at[wid], idx_v)
        pltpu.sync_copy(ones_hbm, ones_v)

        plsc.subcore_barrier()

        row0 = wid * ROWS_PER_W

        def step(j, carry):
            pltpu.sync_copy(x_hbm.at[pl.ds(row0 + j * CHUNK, CHUNK), :], rows_v)
            pltpu.sync_copy(rows_v, sums_sh.at[idx_v.at[j]], add=True)
            pltpu.sync_copy(ones_v, cnts_sh.at[idx_v.at[j]], add=True)
            return carry

        lax.fori_loop(0, CHUNKS_PER_W, step, 0)

        plsc.subcore_barrier()

        pltpu.sync_copy(sums_sh.at[pl.ds(seg0, SEG_PER_TILE), :],
                        psums_hbm.at[cid, pl.ds(seg0, SEG_PER_TILE), :])
        pltpu.sync_copy(cnts_sh.at[pl.ds(seg0, SEG_PER_TILE), :],
                        pcnts_hbm.at[cid, pl.ds(seg0, SEG_PER_TILE), :])

    return pl.kernel(
        body,
        out_type=(
            jax.ShapeDtypeStruct((NC, NUM_SEG, D), jnp.float32),
            jax.ShapeDtypeStruct((NC, NUM_SEG, CNT_W), jnp.float32),
        ),
        mesh=mesh,
        scratch_types=[
            pltpu.VMEM((CHUNKS_PER_W, CHUNK), jnp.int32),
            pltpu.VMEM((CHUNK, D), jnp.float32),
            pltpu.VMEM((CHUNK, CNT_W), jnp.float32),
            pltpu.VMEM_SHARED((NUM_SEG, D), jnp.float32),
            pltpu.VMEM_SHARED((NUM_SEG, CNT_W), jnp.float32),
        ],
    )(x, idx2d, zeros, zeros_c, ones)


def _combine(psums_ref, pcnts_ref, out_ref):
    s = psums_ref[0] + psums_ref[1]
    c = pcnts_ref[0] + pcnts_ref[1]
    denom = jnp.maximum(c[:, :1], 1.0)
    out_ref[...] = s / denom


def kernel(x, batch):
    idx2d = batch.astype(jnp.int32).reshape(NW, CHUNKS_PER_W, CHUNK)
    zeros = jnp.zeros((NUM_SEG, D), jnp.float32)
    zeros_c = jnp.zeros((NUM_SEG, CNT_W), jnp.float32)
    ones = jnp.ones((CHUNK, CNT_W), jnp.float32)
    psums, pcnts = _sc_partial_sums(x, idx2d, zeros, zeros_c, ones)
    return pl.pallas_call(
        _combine,
        out_shape=jax.ShapeDtypeStruct((NUM_SEG, D), jnp.float32),
    )(psums, pcnts)


# speedup vs baseline: 7.8893x; 1.4399x over previous
import jax
import jax.numpy as jnp
from jax import lax
from jax.experimental import pallas as pl
from jax.experimental.pallas import tpu as pltpu
from jax.experimental.pallas import tpu_sc as plsc

N_ROWS = 320000
D = 128
NUM_SEG = 1024
NC = 2
NS = 16
NW = NC * NS
ROWS_PER_W = N_ROWS // NW
CHUNK = 80
CHUNKS_PER_W = ROWS_PER_W // CHUNK
SEG_PER_TILE = NUM_SEG // NS
CNT_W = 16


def _sc_partial_sums(x, idx2d, zeros, zeros_c, ones):
    mesh = plsc.VectorSubcoreMesh(core_axis_name="c", subcore_axis_name="s")

    def body(x_hbm, idx_hbm, zeros_hbm, zeros_c_hbm, ones_hbm, psums_hbm,
             pcnts_hbm, idx_v, rows_v, ones_v, sums_sh, cnts_sh, sems):
        cid = lax.axis_index("c")
        sid = lax.axis_index("s")
        wid = cid * NS + sid

        seg0 = sid * SEG_PER_TILE
        pltpu.sync_copy(zeros_hbm.at[pl.ds(seg0, SEG_PER_TILE), :],
                        sums_sh.at[pl.ds(seg0, SEG_PER_TILE), :])
        pltpu.sync_copy(zeros_c_hbm.at[pl.ds(seg0, SEG_PER_TILE), :],
                        cnts_sh.at[pl.ds(seg0, SEG_PER_TILE), :])

        pltpu.sync_copy(idx_hbm.at[wid], idx_v)
        pltpu.sync_copy(ones_hbm, ones_v)

        plsc.subcore_barrier()

        row0 = wid * ROWS_PER_W

        def load(j, b):
            return pltpu.make_async_copy(
                x_hbm.at[pl.ds(row0 + j * CHUNK, CHUNK), :],
                rows_v.at[b],
                sems.at[b])

        load(0, 0).start()

        def step(j, carry):
            b = lax.rem(j, 2)
            nb = 1 - b

            @pl.when(j + 1 < CHUNKS_PER_W)
            def _():
                load(j + 1, nb).start()

            load(j, b).wait()
            pltpu.sync_copy(rows_v.at[b], sums_sh.at[idx_v.at[j]], add=True)
            pltpu.sync_copy(ones_v, cnts_sh.at[idx_v.at[j]], add=True)
            return carry

        lax.fori_loop(0, CHUNKS_PER_W, step, 0)

        plsc.subcore_barrier()

        pltpu.sync_copy(sums_sh.at[pl.ds(seg0, SEG_PER_TILE), :],
                        psums_hbm.at[cid, pl.ds(seg0, SEG_PER_TILE), :])
        pltpu.sync_copy(cnts_sh.at[pl.ds(seg0, SEG_PER_TILE), :],
                        pcnts_hbm.at[cid, pl.ds(seg0, SEG_PER_TILE), :])

    return pl.kernel(
        body,
        out_type=(
            jax.ShapeDtypeStruct((NC, NUM_SEG, D), jnp.float32),
            jax.ShapeDtypeStruct((NC, NUM_SEG, CNT_W), jnp.float32),
        ),
        mesh=mesh,
        scratch_types=[
            pltpu.VMEM((CHUNKS_PER_W, CHUNK), jnp.int32),
            pltpu.VMEM((2, CHUNK, D), jnp.float32),
            pltpu.VMEM((CHUNK, CNT_W), jnp.float32),
            pltpu.VMEM_SHARED((NUM_SEG, D), jnp.float32),
            pltpu.VMEM_SHARED((NUM_SEG, CNT_W), jnp.float32),
            pltpu.SemaphoreType.DMA((2,)),
        ],
    )(x, idx2d, zeros, zeros_c, ones)


def _combine(psums_ref, pcnts_ref, out_ref):
    s = psums_ref[0] + psums_ref[1]
    c = pcnts_ref[0] + pcnts_ref[1]
    denom = jnp.maximum(c[:, :1], 1.0)
    out_ref[...] = s / denom


def kernel(x, batch):
    idx2d = batch.astype(jnp.int32).reshape(NW, CHUNKS_PER_W, CHUNK)
    zeros = jnp.zeros((NUM_SEG, D), jnp.float32)
    zeros_c = jnp.zeros((NUM_SEG, CNT_W), jnp.float32)
    ones = jnp.ones((CHUNK, CNT_W), jnp.float32)
    psums, pcnts = _sc_partial_sums(x, idx2d, zeros, zeros_c, ones)
    return pl.pallas_call(
        _combine,
        out_shape=jax.ShapeDtypeStruct((NUM_SEG, D), jnp.float32),
    )(psums, pcnts)


# speedup vs baseline: 8.5658x; 1.0857x over previous
import jax
import jax.numpy as jnp
from jax import lax
from jax.experimental import pallas as pl
from jax.experimental.pallas import tpu as pltpu
from jax.experimental.pallas import tpu_sc as plsc

N_ROWS = 320000
D = 128
NUM_SEG = 1024
NC = 2
NS = 16
NW = NC * NS
ROWS_PER_W = N_ROWS // NW
CHUNK = 80
CHUNKS_PER_W = ROWS_PER_W // CHUNK
SEG_PER_TILE = NUM_SEG // NS
L = 16
WINDOWS_PER_W = ROWS_PER_W // L


def _sc_partial_sums(x, idx3d, idx_flat, zeros):
    mesh = plsc.VectorSubcoreMesh(core_axis_name="c", subcore_axis_name="s")

    def body(x_hbm, idx_hbm, idxf_hbm, zeros_hbm, psums_hbm, phists_hbm,
             idx_v, idxf_v, rows_v0, rows_v1, hist2d_v, hist_v, sums_sh,
             sems, ssem):
        cid = lax.axis_index("c")
        sid = lax.axis_index("s")
        wid = cid * NS + sid

        seg0 = sid * SEG_PER_TILE
        pltpu.sync_copy(zeros_hbm.at[pl.ds(seg0, SEG_PER_TILE), :],
                        sums_sh.at[pl.ds(seg0, SEG_PER_TILE), :])

        pltpu.sync_copy(idx_hbm.at[wid], idx_v)
        pltpu.sync_copy(idxf_hbm.at[pl.ds(wid * ROWS_PER_W, ROWS_PER_W)],
                        idxf_v)

        fz = jnp.zeros((L,), jnp.float32)

        def zstep(i, carry):
            for r in range(L):
                hist2d_v[r, pl.ds(i * L, L)] = fz
            return carry

        lax.fori_loop(0, NUM_SEG // L, zstep, 0)

        plsc.subcore_barrier()

        row0 = wid * ROWS_PER_W
        bufs = (rows_v0, rows_v1)

        def load(j, b):
            return pltpu.make_async_copy(
                x_hbm.at[pl.ds(row0 + j * CHUNK, CHUNK), :],
                bufs[b],
                sems.at[b])

        def scatter(j, b):
            d = pltpu.make_async_copy(bufs[b], sums_sh.at[idx_v.at[j]], ssem)
            d.start(add=True)
            d.wait()

        load(0, 0).start()

        def step(t, carry):
            j = t * 2
            load(j + 1, 1).start()
            load(j, 0).wait()
            scatter(j, 0)
            load(j + 2, 0).start()
            load(j + 1, 1).wait()
            scatter(j + 1, 1)
            return carry

        lax.fori_loop(0, (CHUNKS_PER_W - 1) // 2, step, 0)
        load(CHUNKS_PER_W - 1, 0).wait()
        scatter(CHUNKS_PER_W - 1, 0)

        lanes = lax.iota(jnp.int32, L)
        ones_f = jnp.ones((L,), jnp.float32)

        def hstep(k, carry):
            w = idxf_v[pl.ds(k * L, L)]
            plsc.addupdate_scatter(hist2d_v, [lanes, w], ones_f)
            return carry

        lax.fori_loop(0, WINDOWS_PER_W, hstep, 0)

        def rstep(g, carry):
            acc = hist2d_v[0, pl.ds(g * L, L)]
            for r in range(1, L):
                acc = acc + hist2d_v[r, pl.ds(g * L, L)]
            hist_v[pl.ds(g * L, L)] = acc
            return carry

        lax.fori_loop(0, NUM_SEG // L, rstep, 0)

        pltpu.sync_copy(hist_v, phists_hbm.at[cid, sid])

        plsc.subcore_barrier()

        pltpu.sync_copy(sums_sh.at[pl.ds(seg0, SEG_PER_TILE), :],
                        psums_hbm.at[cid, pl.ds(seg0, SEG_PER_TILE), :])

    return pl.kernel(
        body,
        out_type=(
            jax.ShapeDtypeStruct((NC, NUM_SEG, D), jnp.float32),
            jax.ShapeDtypeStruct((NC, NS, NUM_SEG), jnp.float32),
        ),
        mesh=mesh,
        compiler_params=pltpu.CompilerParams(needs_layout_passes=False),
        scratch_types=[
            pltpu.VMEM((CHUNKS_PER_W, CHUNK), jnp.int32),
            pltpu.VMEM((ROWS_PER_W,), jnp.int32),
            pltpu.VMEM((CHUNK, D), jnp.float32),
            pltpu.VMEM((CHUNK, D), jnp.float32),
            pltpu.VMEM((L, NUM_SEG), jnp.float32),
            pltpu.VMEM((NUM_SEG,), jnp.float32),
            pltpu.VMEM_SHARED((NUM_SEG, D), jnp.float32),
            pltpu.SemaphoreType.DMA((2,)),
            pltpu.SemaphoreType.DMA,
        ],
    )(x, idx3d, idx_flat, zeros)


def _combine(psums_ref, phists_ref, out_ref):
    s = psums_ref[0] + psums_ref[1]
    c = jnp.sum(phists_ref[...], axis=(0, 1))
    denom = jnp.maximum(c, 1.0).reshape(NUM_SEG, 1)
    out_ref[...] = s / denom


def kernel(x, batch):
    idx_flat = batch.astype(jnp.int32)
    idx3d = idx_flat.reshape(NW, CHUNKS_PER_W, CHUNK)
    zeros = jnp.zeros((NUM_SEG, D), jnp.float32)
    psums, phists = _sc_partial_sums(x, idx3d, idx_flat, zeros)
    return pl.pallas_call(
        _combine,
        out_shape=jax.ShapeDtypeStruct((NUM_SEG, D), jnp.float32),
    )(psums, phists)


# speedup vs baseline: 9.0515x; 1.0567x over previous
import jax
import jax.numpy as jnp
from jax import lax
from jax.experimental import pallas as pl
from jax.experimental.pallas import tpu as pltpu
from jax.experimental.pallas import tpu_sc as plsc

N_ROWS = 320000
D = 128
NUM_SEG = 1024
NC = 2
NS = 16
NW = NC * NS
ROWS_PER_W = N_ROWS // NW
CHUNK = 80
CHUNKS_PER_W = ROWS_PER_W // CHUNK
NBUF = 4
SEG_PER_TILE = NUM_SEG // NS
L = 16
WIN_PER_CHUNK = CHUNK // L


def _sc_partial_sums(x, idx3d, zeros):
    mesh = plsc.VectorSubcoreMesh(core_axis_name="c", subcore_axis_name="s")

    def body(x_hbm, idx_hbm, zeros_hbm, psums_hbm, phists_hbm,
             idx_v, rows_v0, rows_v1, rows_v2, rows_v3, hist2d_v, sums_sh,
             sems, ssem):
        cid = lax.axis_index("c")
        sid = lax.axis_index("s")
        wid = cid * NS + sid

        seg0 = sid * SEG_PER_TILE
        pltpu.sync_copy(zeros_hbm.at[pl.ds(seg0, SEG_PER_TILE), :],
                        sums_sh.at[pl.ds(seg0, SEG_PER_TILE), :])

        pltpu.sync_copy(idx_hbm.at[wid], idx_v)

        fz = jnp.zeros((L,), jnp.float32)

        def zstep(i, carry):
            for r in range(L):
                hist2d_v[r, pl.ds(i * L, L)] = fz
            return carry

        lax.fori_loop(0, NUM_SEG // L, zstep, 0)

        plsc.subcore_barrier()

        row0 = wid * ROWS_PER_W
        bufs = (rows_v0, rows_v1, rows_v2, rows_v3)
        lanes = lax.iota(jnp.int32, L)
        ones_f = jnp.ones((L,), jnp.float32)

        def load(j, u):
            return pltpu.make_async_copy(
                x_hbm.at[pl.ds(row0 + j * CHUNK, CHUNK), :],
                bufs[u],
                sems.at[u])

        def hwindow(j, wi):
            w = idx_v[j, pl.ds(wi * L, L)]
            plsc.addupdate_scatter(hist2d_v, [lanes, w], ones_f)

        def consume(j, u):
            load(j, u).wait()
            d = pltpu.make_async_copy(bufs[u], sums_sh.at[idx_v.at[j]], ssem)
            d.start(add=True)
            for wi in range(WIN_PER_CHUNK):
                hwindow(j, wi)
            d.wait()

        for u in range(NBUF - 1):
            load(u, u).start()

        MAIN = (CHUNKS_PER_W - (NBUF + 1)) // NBUF * NBUF

        def step(t4, carry):
            for u in range(NBUF):
                j = t4 * NBUF + u
                consume(j, u)
                load(j + NBUF - 1, u - 1 if u else NBUF - 1).start()
            return carry

        lax.fori_loop(0, MAIN // NBUF, step, 0)
        for j in range(MAIN, CHUNKS_PER_W):
            consume(j, j % NBUF)
            if j + NBUF - 1 < CHUNKS_PER_W:
                load(j + NBUF - 1, (j - 1) % NBUF).start()

        def rstep(g, carry):
            acc = hist2d_v[0, pl.ds(g * L, L)]
            for r in range(1, L):
                acc = acc + hist2d_v[r, pl.ds(g * L, L)]
            hist2d_v[0, pl.ds(g * L, L)] = acc
            return carry

        lax.fori_loop(0, NUM_SEG // L, rstep, 0)

        pltpu.sync_copy(hist2d_v.at[0], phists_hbm.at[cid, sid])

        plsc.subcore_barrier()

        pltpu.sync_copy(sums_sh.at[pl.ds(seg0, SEG_PER_TILE), :],
                        psums_hbm.at[cid, pl.ds(seg0, SEG_PER_TILE), :])

    return pl.kernel(
        body,
        out_type=(
            jax.ShapeDtypeStruct((NC, NUM_SEG, D), jnp.float32),
            jax.ShapeDtypeStruct((NC, NS, NUM_SEG), jnp.float32),
        ),
        mesh=mesh,
        compiler_params=pltpu.CompilerParams(needs_layout_passes=False),
        scratch_types=[
            pltpu.VMEM((CHUNKS_PER_W, CHUNK), jnp.int32),
            pltpu.VMEM((CHUNK, D), jnp.float32),
            pltpu.VMEM((CHUNK, D), jnp.float32),
            pltpu.VMEM((CHUNK, D), jnp.float32),
            pltpu.VMEM((CHUNK, D), jnp.float32),
            pltpu.VMEM((L, NUM_SEG), jnp.float32),
            pltpu.VMEM_SHARED((NUM_SEG, D), jnp.float32),
            pltpu.SemaphoreType.DMA((NBUF,)),
            pltpu.SemaphoreType.DMA,
        ],
    )(x, idx3d, zeros)


def _combine(psums_ref, phists_ref, out_ref):
    s = psums_ref[0] + psums_ref[1]
    c = jnp.sum(phists_ref[...], axis=(0, 1))
    denom = jnp.maximum(c, 1.0).reshape(NUM_SEG, 1)
    out_ref[...] = s / denom


def kernel(x, batch):
    idx_flat = batch.astype(jnp.int32)
    idx3d = idx_flat.reshape(NW, CHUNKS_PER_W, CHUNK)
    zeros = jnp.zeros((NUM_SEG, D), jnp.float32)
    psums, phists = _sc_partial_sums(x, idx3d, zeros)
    return pl.pallas_call(
        _combine,
        out_shape=jax.ShapeDtypeStruct((NUM_SEG, D), jnp.float32),
    )(psums, phists)


# speedup vs baseline: 9.6335x; 1.0643x over previous
import jax
import jax.numpy as jnp
from jax import lax
from jax.experimental import pallas as pl
from jax.experimental.pallas import tpu as pltpu
from jax.experimental.pallas import tpu_sc as plsc

N_ROWS = 320000
D = 128
NUM_SEG = 1024
NC = 2
NS = 16
NW = NC * NS
ROWS_PER_W = N_ROWS // NW
CHUNK = 128
CHUNKS_PER_W = -(-ROWS_PER_W // CHUNK)
TAIL = ROWS_PER_W - (CHUNKS_PER_W - 1) * CHUNK
PAD = CHUNKS_PER_W * CHUNK - ROWS_PER_W
NBUF = 4
SEG_PER_TILE = NUM_SEG // NS
L = 16
WIN_PER_CHUNK = CHUNK // L
HIST_W = NUM_SEG + L
ACC_ROWS = NUM_SEG + 8


def _sc_partial_sums(x, idx3d, zeros):
    mesh = plsc.VectorSubcoreMesh(core_axis_name="c", subcore_axis_name="s")

    def body(x_hbm, idx_hbm, zeros_hbm, psums_hbm, phists_hbm,
             idx_v, rows_v0, rows_v1, rows_v2, rows_v3, hist2d_v, sums_sh,
             sems, ssem):
        cid = lax.axis_index("c")
        sid = lax.axis_index("s")
        wid = cid * NS + sid

        seg0 = sid * SEG_PER_TILE
        pltpu.sync_copy(zeros_hbm.at[pl.ds(seg0, SEG_PER_TILE), :],
                        sums_sh.at[pl.ds(seg0, SEG_PER_TILE), :])

        pltpu.sync_copy(idx_hbm.at[wid], idx_v)

        fz = jnp.zeros((L,), jnp.float32)

        def zstep(i, carry):
            for r in range(L):
                hist2d_v[r, pl.ds(i * L, L)] = fz
            return carry

        lax.fori_loop(0, NUM_SEG // L, zstep, 0)

        plsc.subcore_barrier()

        row0 = wid * ROWS_PER_W
        bufs = (rows_v0, rows_v1, rows_v2, rows_v3)
        lanes = lax.iota(jnp.int32, L)
        ones_f = jnp.ones((L,), jnp.float32)

        def load(j, u, n=CHUNK):
            return pltpu.make_async_copy(
                x_hbm.at[pl.ds(row0 + j * CHUNK, n)],
                bufs[u].at[pl.ds(0, n)],
                sems.at[u])

        def consume(j, u, n=CHUNK):
            load(j, u, n).wait()
            d = pltpu.make_async_copy(bufs[u], sums_sh.at[idx_v.at[j]], ssem)
            d.start(add=True)
            for wi in range(WIN_PER_CHUNK):
                w = idx_v[j, pl.ds(wi * L, L)]
                plsc.addupdate_scatter(hist2d_v, [lanes, w], ones_f)
            d.wait()

        for u in range(NBUF - 1):
            load(u, u).start()

        MAIN = (CHUNKS_PER_W - (NBUF + 1)) // NBUF * NBUF

        def step(t4, carry):
            for u in range(NBUF):
                j = t4 * NBUF + u
                consume(j, u)
                load(j + NBUF - 1, u - 1 if u else NBUF - 1).start()
            return carry

        lax.fori_loop(0, MAIN // NBUF, step, 0)
        for j in range(MAIN, CHUNKS_PER_W):
            consume(j, j % NBUF, CHUNK if j < CHUNKS_PER_W - 1 else TAIL)
            jn = j + NBUF - 1
            if jn < CHUNKS_PER_W:
                load(jn, (j - 1) % NBUF,
                     CHUNK if jn < CHUNKS_PER_W - 1 else TAIL).start()

        def rstep(g, carry):
            acc = hist2d_v[0, pl.ds(g * L, L)]
            for r in range(1, L):
                acc = acc + hist2d_v[r, pl.ds(g * L, L)]
            hist2d_v[0, pl.ds(g * L, L)] = acc
            return carry

        lax.fori_loop(0, NUM_SEG // L, rstep, 0)

        pltpu.sync_copy(hist2d_v.at[0, pl.ds(0, NUM_SEG)],
                        phists_hbm.at[cid, sid])

        plsc.subcore_barrier()

        pltpu.sync_copy(sums_sh.at[pl.ds(seg0, SEG_PER_TILE), :],
                        psums_hbm.at[cid, pl.ds(seg0, SEG_PER_TILE), :])

    return pl.kernel(
        body,
        out_type=(
            jax.ShapeDtypeStruct((NC, NUM_SEG, D), jnp.float32),
            jax.ShapeDtypeStruct((NC, NS, NUM_SEG), jnp.float32),
        ),
        mesh=mesh,
        compiler_params=pltpu.CompilerParams(needs_layout_passes=False),
        scratch_types=[
            pltpu.VMEM((CHUNKS_PER_W, CHUNK), jnp.int32),
            pltpu.VMEM((CHUNK, D), jnp.float32),
            pltpu.VMEM((CHUNK, D), jnp.float32),
            pltpu.VMEM((CHUNK, D), jnp.float32),
            pltpu.VMEM((CHUNK, D), jnp.float32),
            pltpu.VMEM((L, HIST_W), jnp.float32),
            pltpu.VMEM_SHARED((ACC_ROWS, D), jnp.float32),
            pltpu.SemaphoreType.DMA((NBUF,)),
            pltpu.SemaphoreType.DMA,
        ],
    )(x, idx3d, zeros)


def _combine(psums_ref, phists_ref, out_ref):
    s = psums_ref[0] + psums_ref[1]
    c = jnp.sum(phists_ref[...], axis=(0, 1))
    denom = jnp.maximum(c, 1.0).reshape(NUM_SEG, 1)
    out_ref[...] = s / denom


def kernel(x, batch):
    idx_flat = batch.astype(jnp.int32)
    idx_pad = jnp.concatenate(
        [idx_flat.reshape(NW, ROWS_PER_W),
         jnp.full((NW, PAD), NUM_SEG, jnp.int32)], axis=1)
    idx3d = idx_pad.reshape(NW, CHUNKS_PER_W, CHUNK)
    zeros = jnp.zeros((NUM_SEG, D), jnp.float32)
    psums, phists = _sc_partial_sums(x, idx3d, zeros)
    return pl.pallas_call(
        _combine,
        out_shape=jax.ShapeDtypeStruct((NUM_SEG, D), jnp.float32),
    )(psums, phists)
